# baseline (speedup 1.0000x reference)
import functools

import jax
import jax.numpy as jnp
from jax import lax
from jax.experimental import pallas as pl
from jax.experimental.pallas import tpu as pltpu
from jax.experimental.pallas import tpu_sc as plsc

N = 10000
E = 320000
LANES = 16
NS = 16
NC = 2
CHUNK = 100
BM = 2000

_MESH = plsc.VectorSubcoreMesh(core_axis_name="c", subcore_axis_name="s")
_SC_PARAMS = pltpu.CompilerParams(use_tc_tiling_on_sc=False)


def _sc_degree(dst_r, zeros, ones):
    nchunks = E // (NC * NS * CHUNK)

    @functools.partial(
        pl.kernel,
        out_type=jax.ShapeDtypeStruct((NC, N, LANES), jnp.float32),
        mesh=_MESH,
        scratch_types=[
            pltpu.VMEM((nchunks, CHUNK), jnp.int32),
            pltpu.VMEM((CHUNK, LANES), jnp.float32),
            pltpu.VMEM_SHARED((N, LANES), jnp.float32),
            pltpu.SemaphoreType.DMA,
            pltpu.SemaphoreType.DMA,
        ],
        compiler_params=_SC_PARAMS,
    )
    def k(dst_hbm, zeros_hbm, ones_hbm, out_hbm, bigdst, onesbuf, acc,
          sem_a, sem_b):
        c = lax.axis_index("c")
        s = lax.axis_index("s")
        t = c * NS + s
        pltpu.sync_copy(dst_hbm.at[t], bigdst)
        pltpu.sync_copy(ones_hbm, onesbuf)

        @pl.when(s == 0)
        def _():
            pltpu.sync_copy(zeros_hbm, acc)

        plsc.subcore_barrier()

        def issue(j, sem):
            pltpu.async_copy(onesbuf, acc.at[bigdst.at[j]], sem, add=True)

        def wait(j, sem):
            pltpu.make_async_copy(onesbuf, acc.at[bigdst.at[j]], sem).wait()

        issue(0, sem_a)

        def step(i, carry):
            j = 2 * i
            issue(j + 1, sem_b)
            wait(j, sem_a)

            @pl.when(j + 2 < nchunks)
            def _():
                issue(j + 2, sem_a)

            wait(j + 1, sem_b)
            return carry

        lax.fori_loop(0, nchunks // 2, step, 0)
        plsc.subcore_barrier()

        @pl.when(s == 0)
        def _():
            pltpu.sync_copy(acc, out_hbm.at[c])

    return k(dst_r, zeros, ones)


def _sc_aggregate(h0, h1, srcpk, dstpk, fh, chunk):
    n = E // (NS * chunk)

    @functools.partial(
        pl.kernel,
        out_type=(
            jax.ShapeDtypeStruct((N, fh), jnp.bfloat16),
            jax.ShapeDtypeStruct((N, fh), jnp.bfloat16),
        ),
        mesh=_MESH,
        scratch_types=[
            pltpu.VMEM((4, chunk), jnp.int32),
            pltpu.VMEM((8, chunk), jnp.int32),
            pltpu.VMEM((4, chunk, fh), jnp.bfloat16),
            pltpu.VMEM_SHARED((N, fh), jnp.bfloat16),
        ] + [pltpu.SemaphoreType.DMA] * 24,
        compiler_params=_SC_PARAMS,
    )
    def k(h0_hbm, h1_hbm, src_hbm, dst_hbm, a0_hbm, a1_hbm,
          sidx, didx, g, acc, *sems):
        s_g = sems[0:4]
        s_s = sems[4:12]
        s_r = sems[12:16]
        s_d = sems[16:24]
        c = lax.axis_index("c")
        s = lax.axis_index("s")

        def issue_gather(kk):
            @pl.when(c == 0)
            def _():
                pltpu.async_copy(h0_hbm.at[sidx.at[kk]], g.at[kk], s_g[kk])

            @pl.when(c == 1)
            def _():
                pltpu.async_copy(h1_hbm.at[sidx.at[kk]], g.at[kk], s_g[kk])

        def wait_gather(kk):
            pltpu.make_async_copy(h0_hbm.at[sidx.at[kk]], g.at[kk],
                                  s_g[kk]).wait()

        def issue_src(j, kk):
            pltpu.async_copy(src_hbm.at[s, j], sidx.at[kk], s_r[kk])

        def wait_src(kk):
            pltpu.make_async_copy(src_hbm.at[s, 0], sidx.at[kk],
                                  s_r[kk]).wait()

        def issue_dst(j, kk):
            pltpu.async_copy(dst_hbm.at[s, j], didx.at[kk], s_d[kk])

        def wait_dst(kk):
            pltpu.make_async_copy(dst_hbm.at[s, 0], didx.at[kk],
                                  s_d[kk]).wait()

        def issue_scatter(kg, kk):
            pltpu.async_copy(g.at[kg], acc.at[didx.at[kk]], s_s[kk], add=True)

        def wait_scatter(kg, kk):
            pltpu.make_async_copy(g.at[kg], acc.at[didx.at[kk]],
                                  s_s[kk]).wait()

        rows = N // NS
        sl = pl.ds(s * rows, rows)

        @pl.when(c == 0)
        def _():
            pltpu.sync_copy(h0_hbm.at[sl], acc.at[sl])

        @pl.when(c == 1)
        def _():
            pltpu.sync_copy(h1_hbm.at[sl], acc.at[sl])

        for kk in range(4):
            issue_src(kk, kk)
        for kk in range(6):
            issue_dst(kk, kk)
        for kk in range(2):
            wait_src(kk)
            issue_gather(kk)
        plsc.subcore_barrier()

        def chunk_body(i, kk, first, last):
            j = 8 * i + kk
            kg = kk % 4
            wait_gather(kg)
            wait_dst(kk)
            issue_scatter(kg, kk)
            if not (first and kk < 2):
                wait_scatter((kg + 2) % 4, (kk + 6) % 8)
            if not last or kk < 2:
                issue_dst(j + 6, (kk + 6) % 8)
            if not last or kk < 4:
                issue_src(j + 4, kg)
            if not last or kk < 6:
                wait_src((kg + 2) % 4)
                issue_gather((kg + 2) % 4)

        for kk in range(8):
            chunk_body(0, kk, True, False)

        def step(i, carry):
            for kk in range(8):
                chunk_body(i, kk, False, False)
            return carry

        lax.fori_loop(1, n // 8 - 1, step, 0)
        for kk in range(8):
            chunk_body(n // 8 - 1, kk, False, True)
        wait_scatter(2, 6)
        wait_scatter(3, 7)
        plsc.subcore_barrier()

        @pl.when(c == 0)
        def _():
            pltpu.sync_copy(acc.at[sl], a0_hbm.at[sl])

        @pl.when(c == 1)
        def _():
            pltpu.sync_copy(acc.at[sl], a1_hbm.at[sl])

    return k(h0, h1, srcpk, dstpk)


def _tc_mm1_fused(x, W1, degpart):
    def body(x_ref, w_ref, p_ref, d_ref, o0_ref, o1_ref):
        p = p_ref[...]
        dinv = lax.rsqrt(p[0] + p[1] + 1.0)
        d_ref[...] = dinv
        h = jnp.dot(x_ref[...], w_ref[...], preferred_element_type=jnp.float32)
        hp = (dinv[:, 0:1] * h).astype(jnp.bfloat16)
        o0_ref[...] = hp[:, :128]
        o1_ref[...] = hp[:, 128:]

    return pl.pallas_call(
        body,
        grid=(N // BM,),
        out_shape=(
            jax.ShapeDtypeStruct((N, LANES), jnp.float32),
            jax.ShapeDtypeStruct((N, 128), jnp.bfloat16),
            jax.ShapeDtypeStruct((N, 128), jnp.bfloat16),
        ),
        in_specs=[
            pl.BlockSpec((BM, 128), lambda i: (i, 0)),
            pl.BlockSpec((128, 256), lambda i: (0, 0)),
            pl.BlockSpec((NC, BM, LANES), lambda i: (0, i, 0)),
        ],
        out_specs=(
            pl.BlockSpec((BM, LANES), lambda i: (i, 0)),
            pl.BlockSpec((BM, 128), lambda i: (i, 0)),
            pl.BlockSpec((BM, 128), lambda i: (i, 0)),
        ),
    )(x, W1, degpart)


def _tc_mm_next(a0, a1, bprev, W, dinv16, fout):
    fh = fout // 2

    def body(a0_ref, a1_ref, b_ref, w_ref, d_ref, o0_ref, o1_ref):
        d1 = d_ref[:, 0:1]
        xin = jnp.concatenate(
            [a0_ref[...], a1_ref[...]], axis=1).astype(jnp.float32)
        pre = d1 * xin + b_ref[...]
        xact = jnp.where(pre > 0, pre, 0.1 * pre)
        h = jnp.dot(xact, w_ref[...], preferred_element_type=jnp.float32)
        hp = (d1 * h).astype(jnp.bfloat16)
        o0_ref[...] = hp[:, :fh]
        o1_ref[...] = hp[:, fh:]

    return pl.pallas_call(
        body,
        grid=(N // BM,),
        out_shape=(
            jax.ShapeDtypeStruct((N, fh), jnp.bfloat16),
            jax.ShapeDtypeStruct((N, fh), jnp.bfloat16),
        ),
        in_specs=[
            pl.BlockSpec((BM, 128), lambda i: (i, 0)),
            pl.BlockSpec((BM, 128), lambda i: (i, 0)),
            pl.BlockSpec((1, 256), lambda i: (0, 0)),
            pl.BlockSpec((256, fout), lambda i: (0, 0)),
            pl.BlockSpec((BM, LANES), lambda i: (i, 0)),
        ],
        out_specs=(
            pl.BlockSpec((BM, fh), lambda i: (i, 0)),
            pl.BlockSpec((BM, fh), lambda i: (i, 0)),
        ),
    )(a0, a1, bprev, W, dinv16)


def _tc_final(a0, a1, b3, dinv16):
    def body(a0_ref, a1_ref, b_ref, d_ref, o_ref):
        d1 = d_ref[:, 0:1]
        xin = jnp.concatenate(
            [a0_ref[...], a1_ref[...]], axis=1).astype(jnp.float32)
        o_ref[...] = d1 * xin + b_ref[...]

    return pl.pallas_call(
        body,
        grid=(N // BM,),
        out_shape=jax.ShapeDtypeStruct((N, 128), jnp.float32),
        in_specs=[
            pl.BlockSpec((BM, 64), lambda i: (i, 0)),
            pl.BlockSpec((BM, 64), lambda i: (i, 0)),
            pl.BlockSpec((1, 128), lambda i: (0, 0)),
            pl.BlockSpec((BM, LANES), lambda i: (i, 0)),
        ],
        out_specs=pl.BlockSpec((BM, 128), lambda i: (i, 0)),
    )(a0, a1, b3, dinv16)


def kernel(x, edge_index, W1, b1, W2, b2, W3, b3):
    src = edge_index[0].astype(jnp.int32)
    dst = edge_index[1].astype(jnp.int32)
    dst_deg = dst.reshape(NC * NS, E // (NC * NS * CHUNK), CHUNK)
    agg_chunk = 100
    nagg = E // (NS * agg_chunk)
    srcpk = src.reshape(NS, nagg, agg_chunk)
    dstpk = dst.reshape(NS, nagg, agg_chunk)
    srcpk3 = src.reshape(NS, 160, 125)
    dstpk3 = dst.reshape(NS, 160, 125)
    zeros = jnp.zeros((N, LANES), jnp.float32)
    ones = jnp.ones((CHUNK, LANES), jnp.float32)

    degpart = _sc_degree(dst_deg, zeros, ones)
    dinv16, h0, h1 = _tc_mm1_fused(x, W1, degpart)
    a0, a1 = _sc_aggregate(h0, h1, srcpk, dstpk, 128, agg_chunk)
    h0, h1 = _tc_mm_next(a0, a1, b1.reshape(1, 256), W2, dinv16, 256)
    a0, a1 = _sc_aggregate(h0, h1, srcpk, dstpk, 128, agg_chunk)
    h0, h1 = _tc_mm_next(a0, a1, b2.reshape(1, 256), W3, dinv16, 128)
    a0, a1 = _sc_aggregate(h0, h1, srcpk, dstpk, 64, agg_chunk)
    return _tc_final(a0, a1, b3.reshape(1, 128), dinv16)


# speedup vs baseline: 1.1368x; 1.1368x over previous
import functools

import jax
import jax.numpy as jnp
from jax import lax
from jax.experimental import pallas as pl
from jax.experimental.pallas import tpu as pltpu
from jax.experimental.pallas import tpu_sc as plsc

N = 10000
E = 320000
LANES = 16
NS = 16
NC = 2
CHUNK = 100
BM = 2000

_MESH = plsc.VectorSubcoreMesh(core_axis_name="c", subcore_axis_name="s")
_SC_PARAMS = pltpu.CompilerParams(use_tc_tiling_on_sc=False)


def _sc_degree(dst_r, zeros, ones):
    nchunks = E // (NC * NS * CHUNK)

    @functools.partial(
        pl.kernel,
        out_type=jax.ShapeDtypeStruct((NC, N, LANES), jnp.float32),
        mesh=_MESH,
        scratch_types=[
            pltpu.VMEM((nchunks, CHUNK), jnp.int32),
            pltpu.VMEM((CHUNK, LANES), jnp.float32),
            pltpu.VMEM_SHARED((N, LANES), jnp.float32),
            pltpu.SemaphoreType.DMA,
            pltpu.SemaphoreType.DMA,
        ],
        compiler_params=_SC_PARAMS,
    )
    def k(dst_hbm, zeros_hbm, ones_hbm, out_hbm, bigdst, onesbuf, acc,
          sem_a, sem_b):
        c = lax.axis_index("c")
        s = lax.axis_index("s")
        t = c * NS + s
        pltpu.sync_copy(dst_hbm.at[t], bigdst)
        pltpu.sync_copy(ones_hbm, onesbuf)

        @pl.when(s == 0)
        def _():
            pltpu.sync_copy(zeros_hbm, acc)

        plsc.subcore_barrier()

        def issue(j, sem):
            pltpu.async_copy(onesbuf, acc.at[bigdst.at[j]], sem, add=True)

        def wait(j, sem):
            pltpu.make_async_copy(onesbuf, acc.at[bigdst.at[j]], sem).wait()

        issue(0, sem_a)

        def step(i, carry):
            j = 2 * i
            issue(j + 1, sem_b)
            wait(j, sem_a)

            @pl.when(j + 2 < nchunks)
            def _():
                issue(j + 2, sem_a)

            wait(j + 1, sem_b)
            return carry

        lax.fori_loop(0, nchunks // 2, step, 0)
        plsc.subcore_barrier()

        @pl.when(s == 0)
        def _():
            pltpu.sync_copy(acc, out_hbm.at[c])

    return k(dst_r, zeros, ones)


def _sc_aggregate(h0, h1, srcpk, dstpk, fh, chunk):
    n = E // (NS * chunk)

    @functools.partial(
        pl.kernel,
        out_type=(
            jax.ShapeDtypeStruct((N, fh), jnp.bfloat16),
            jax.ShapeDtypeStruct((N, fh), jnp.bfloat16),
        ),
        mesh=_MESH,
        scratch_types=[
            pltpu.VMEM((4, chunk), jnp.int32),
            pltpu.VMEM((4, chunk), jnp.int32),
            pltpu.VMEM((4, chunk, fh), jnp.bfloat16),
            pltpu.VMEM_SHARED((N, fh), jnp.bfloat16),
        ] + [pltpu.SemaphoreType.DMA] * 16,
        compiler_params=_SC_PARAMS,
    )
    def k(h0_hbm, h1_hbm, src_hbm, dst_hbm, a0_hbm, a1_hbm,
          sidx, didx, g, acc, *sems):
        s_g = sems[0:4]
        s_s = sems[4:8]
        s_r = sems[8:12]
        s_d = sems[12:16]
        c = lax.axis_index("c")
        s = lax.axis_index("s")

        def issue_gather(kk):
            @pl.when(c == 0)
            def _():
                pltpu.async_copy(h0_hbm.at[sidx.at[kk]], g.at[kk], s_g[kk])

            @pl.when(c == 1)
            def _():
                pltpu.async_copy(h1_hbm.at[sidx.at[kk]], g.at[kk], s_g[kk])

        def wait_gather(kk):
            pltpu.make_async_copy(h0_hbm.at[sidx.at[kk]], g.at[kk],
                                  s_g[kk]).wait()

        def issue_src(j, kk):
            pltpu.async_copy(src_hbm.at[s, j], sidx.at[kk], s_r[kk])

        def wait_src(kk):
            pltpu.make_async_copy(src_hbm.at[s, 0], sidx.at[kk],
                                  s_r[kk]).wait()

        def issue_dst(j, kk):
            pltpu.async_copy(dst_hbm.at[s, j], didx.at[kk], s_d[kk])

        def wait_dst(kk):
            pltpu.make_async_copy(dst_hbm.at[s, 0], didx.at[kk],
                                  s_d[kk]).wait()

        def issue_scatter(kk):
            pltpu.async_copy(g.at[kk], acc.at[didx.at[kk]], s_s[kk], add=True)

        def wait_scatter(kk):
            pltpu.make_async_copy(g.at[kk], acc.at[didx.at[kk]],
                                  s_s[kk]).wait()

        rows = N // NS
        sl = pl.ds(s * rows, rows)

        @pl.when(c == 0)
        def _():
            pltpu.sync_copy(h0_hbm.at[sl], acc.at[sl])

        @pl.when(c == 1)
        def _():
            pltpu.sync_copy(h1_hbm.at[sl], acc.at[sl])

        for kk in range(4):
            issue_src(kk, kk)
        for kk in range(3):
            issue_dst(kk, kk)
        for kk in range(3):
            wait_src(kk)
            issue_gather(kk)
        plsc.subcore_barrier()

        def chunk_body(i, kk, first, last):
            j = 4 * i + kk
            wait_gather(kk)
            wait_dst(kk)
            issue_scatter(kk)
            if not (first and kk == 0):
                wait_scatter((kk + 3) % 4)
            km = (kk + 3) % 4
            if not last or kk < 1:
                issue_dst(j + 3, km)
            if not last:
                issue_src(j + 4, kk)
            if not last or kk < 1:
                wait_src(km)
                issue_gather(km)

        for kk in range(4):
            chunk_body(0, kk, True, False)

        def step(i, carry):
            for kk in range(4):
                chunk_body(i, kk, False, False)
            return carry

        lax.fori_loop(1, n // 4 - 1, step, 0)
        for kk in range(4):
            chunk_body(n // 4 - 1, kk, False, True)
        wait_scatter(3)
        plsc.subcore_barrier()

        @pl.when(c == 0)
        def _():
            pltpu.sync_copy(acc.at[sl], a0_hbm.at[sl])

        @pl.when(c == 1)
        def _():
            pltpu.sync_copy(acc.at[sl], a1_hbm.at[sl])

    return k(h0, h1, srcpk, dstpk)


def _tc_mm1_fused(x, W1, degpart):
    def body(x_ref, w_ref, p_ref, d_ref, o0_ref, o1_ref):
        p = p_ref[...]
        dinv = lax.rsqrt(p[0] + p[1] + 1.0)
        d_ref[...] = dinv
        h = jnp.dot(x_ref[...], w_ref[...], preferred_element_type=jnp.float32)
        hp = (dinv[:, 0:1] * h).astype(jnp.bfloat16)
        o0_ref[...] = hp[:, :128]
        o1_ref[...] = hp[:, 128:]

    return pl.pallas_call(
        body,
        grid=(N // BM,),
        out_shape=(
            jax.ShapeDtypeStruct((N, LANES), jnp.float32),
            jax.ShapeDtypeStruct((N, 128), jnp.bfloat16),
            jax.ShapeDtypeStruct((N, 128), jnp.bfloat16),
        ),
        in_specs=[
            pl.BlockSpec((BM, 128), lambda i: (i, 0)),
            pl.BlockSpec((128, 256), lambda i: (0, 0)),
            pl.BlockSpec((NC, BM, LANES), lambda i: (0, i, 0)),
        ],
        out_specs=(
            pl.BlockSpec((BM, LANES), lambda i: (i, 0)),
            pl.BlockSpec((BM, 128), lambda i: (i, 0)),
            pl.BlockSpec((BM, 128), lambda i: (i, 0)),
        ),
    )(x, W1, degpart)


def _tc_mm_next(a0, a1, bprev, W, dinv16, fout):
    fh = fout // 2

    def body(a0_ref, a1_ref, b_ref, w_ref, d_ref, o0_ref, o1_ref):
        d1 = d_ref[:, 0:1]
        xin = jnp.concatenate(
            [a0_ref[...], a1_ref[...]], axis=1).astype(jnp.float32)
        pre = d1 * xin + b_ref[...]
        xact = jnp.where(pre > 0, pre, 0.1 * pre)
        h = jnp.dot(xact, w_ref[...], preferred_element_type=jnp.float32)
        hp = (d1 * h).astype(jnp.bfloat16)
        o0_ref[...] = hp[:, :fh]
        o1_ref[...] = hp[:, fh:]

    return pl.pallas_call(
        body,
        grid=(N // BM,),
        out_shape=(
            jax.ShapeDtypeStruct((N, fh), jnp.bfloat16),
            jax.ShapeDtypeStruct((N, fh), jnp.bfloat16),
        ),
        in_specs=[
            pl.BlockSpec((BM, 128), lambda i: (i, 0)),
            pl.BlockSpec((BM, 128), lambda i: (i, 0)),
            pl.BlockSpec((1, 256), lambda i: (0, 0)),
            pl.BlockSpec((256, fout), lambda i: (0, 0)),
            pl.BlockSpec((BM, LANES), lambda i: (i, 0)),
        ],
        out_specs=(
            pl.BlockSpec((BM, fh), lambda i: (i, 0)),
            pl.BlockSpec((BM, fh), lambda i: (i, 0)),
        ),
    )(a0, a1, bprev, W, dinv16)


def _tc_final(a0, a1, b3, dinv16):
    def body(a0_ref, a1_ref, b_ref, d_ref, o_ref):
        d1 = d_ref[:, 0:1]
        xin = jnp.concatenate(
            [a0_ref[...], a1_ref[...]], axis=1).astype(jnp.float32)
        o_ref[...] = d1 * xin + b_ref[...]

    return pl.pallas_call(
        body,
        grid=(N // BM,),
        out_shape=jax.ShapeDtypeStruct((N, 128), jnp.float32),
        in_specs=[
            pl.BlockSpec((BM, 64), lambda i: (i, 0)),
            pl.BlockSpec((BM, 64), lambda i: (i, 0)),
            pl.BlockSpec((1, 128), lambda i: (0, 0)),
            pl.BlockSpec((BM, LANES), lambda i: (i, 0)),
        ],
        out_specs=pl.BlockSpec((BM, 128), lambda i: (i, 0)),
    )(a0, a1, b3, dinv16)


def kernel(x, edge_index, W1, b1, W2, b2, W3, b3):
    src = edge_index[0].astype(jnp.int32)
    dst = edge_index[1].astype(jnp.int32)
    dst_deg = dst.reshape(NC * NS, E // (NC * NS * CHUNK), CHUNK)
    agg_chunk = 100
    nagg = E // (NS * agg_chunk)
    srcpk = src.reshape(NS, nagg, agg_chunk)
    dstpk = dst.reshape(NS, nagg, agg_chunk)
    srcpk3 = src.reshape(NS, 160, 125)
    dstpk3 = dst.reshape(NS, 160, 125)
    zeros = jnp.zeros((N, LANES), jnp.float32)
    ones = jnp.ones((CHUNK, LANES), jnp.float32)

    degpart = _sc_degree(dst_deg, zeros, ones)
    dinv16, h0, h1 = _tc_mm1_fused(x, W1, degpart)
    a0, a1 = _sc_aggregate(h0, h1, srcpk, dstpk, 128, agg_chunk)
    h0, h1 = _tc_mm_next(a0, a1, b1.reshape(1, 256), W2, dinv16, 256)
    a0, a1 = _sc_aggregate(h0, h1, srcpk, dstpk, 128, agg_chunk)
    h0, h1 = _tc_mm_next(a0, a1, b2.reshape(1, 256), W3, dinv16, 128)
    a0, a1 = _sc_aggregate(h0, h1, srcpk, dstpk, 64, agg_chunk)
    return _tc_final(a0, a1, b3.reshape(1, 128), dinv16)


# speedup vs baseline: 1.1384x; 1.0014x over previous
import functools

import jax
import jax.numpy as jnp
from jax import lax
from jax.experimental import pallas as pl
from jax.experimental.pallas import tpu as pltpu
from jax.experimental.pallas import tpu_sc as plsc

N = 10000
E = 320000
LANES = 16
NS = 16
NC = 2
CHUNK = 100
BM = 2000

_MESH = plsc.VectorSubcoreMesh(core_axis_name="c", subcore_axis_name="s")
_SC_PARAMS = pltpu.CompilerParams(use_tc_tiling_on_sc=False)


def _sc_degree(dst_r, zeros, ones):
    nchunks = E // (NC * NS * CHUNK)

    @functools.partial(
        pl.kernel,
        out_type=jax.ShapeDtypeStruct((NC, N, LANES), jnp.float32),
        mesh=_MESH,
        scratch_types=[
            pltpu.VMEM((nchunks, CHUNK), jnp.int32),
            pltpu.VMEM((CHUNK, LANES), jnp.float32),
            pltpu.VMEM_SHARED((N, LANES), jnp.float32),
            pltpu.SemaphoreType.DMA,
            pltpu.SemaphoreType.DMA,
        ],
        compiler_params=_SC_PARAMS,
    )
    def k(dst_hbm, zeros_hbm, ones_hbm, out_hbm, bigdst, onesbuf, acc,
          sem_a, sem_b):
        c = lax.axis_index("c")
        s = lax.axis_index("s")
        t = c * NS + s
        pltpu.sync_copy(dst_hbm.at[t], bigdst)
        pltpu.sync_copy(ones_hbm, onesbuf)

        @pl.when(s == 0)
        def _():
            pltpu.sync_copy(zeros_hbm, acc)

        plsc.subcore_barrier()

        def issue(j, sem):
            pltpu.async_copy(onesbuf, acc.at[bigdst.at[j]], sem, add=True)

        def wait(j, sem):
            pltpu.make_async_copy(onesbuf, acc.at[bigdst.at[j]], sem).wait()

        issue(0, sem_a)

        def step(i, carry):
            j = 2 * i
            issue(j + 1, sem_b)
            wait(j, sem_a)

            @pl.when(j + 2 < nchunks)
            def _():
                issue(j + 2, sem_a)

            wait(j + 1, sem_b)
            return carry

        lax.fori_loop(0, nchunks // 2, step, 0)
        plsc.subcore_barrier()

        @pl.when(s == 0)
        def _():
            pltpu.sync_copy(acc, out_hbm.at[c])

    return k(dst_r, zeros, ones)


def _sc_aggregate(h0, h1, srcpk, dstpk, fh, chunk):
    n = E // (NS * chunk)

    @functools.partial(
        pl.kernel,
        out_type=(
            jax.ShapeDtypeStruct((N, fh), jnp.bfloat16),
            jax.ShapeDtypeStruct((N, fh), jnp.bfloat16),
        ),
        mesh=_MESH,
        scratch_types=[
            pltpu.VMEM((4, chunk), jnp.int32),
            pltpu.VMEM((4, chunk), jnp.int32),
            pltpu.VMEM((4, chunk, fh), jnp.bfloat16),
            pltpu.VMEM_SHARED((N, fh), jnp.bfloat16),
        ] + [pltpu.SemaphoreType.DMA] * 16,
        compiler_params=_SC_PARAMS,
    )
    def k(h0_hbm, h1_hbm, src_hbm, dst_hbm, a0_hbm, a1_hbm,
          sidx, didx, g, acc, *sems):
        s_g = sems[0:4]
        s_s = sems[4:8]
        s_r = sems[8:12]
        s_d = sems[12:16]
        c = lax.axis_index("c")
        s = lax.axis_index("s")

        def issue_gather(kk):
            @pl.when(c == 0)
            def _():
                pltpu.async_copy(h0_hbm.at[sidx.at[kk]], g.at[kk], s_g[kk])

            @pl.when(c == 1)
            def _():
                pltpu.async_copy(h1_hbm.at[sidx.at[kk]], g.at[kk], s_g[kk])

        def wait_gather(kk):
            pltpu.make_async_copy(h0_hbm.at[sidx.at[kk]], g.at[kk],
                                  s_g[kk]).wait()

        def issue_src(j, kk):
            pltpu.async_copy(src_hbm.at[s, j], sidx.at[kk], s_r[kk])

        def wait_src(kk):
            pltpu.make_async_copy(src_hbm.at[s, 0], sidx.at[kk],
                                  s_r[kk]).wait()

        def issue_dst(j, kk):
            pltpu.async_copy(dst_hbm.at[s, j], didx.at[kk], s_d[kk])

        def wait_dst(kk):
            pltpu.make_async_copy(dst_hbm.at[s, 0], didx.at[kk],
                                  s_d[kk]).wait()

        def issue_scatter(kk):
            pltpu.async_copy(g.at[kk], acc.at[didx.at[kk]], s_s[kk], add=True)

        def wait_scatter(kk):
            pltpu.make_async_copy(g.at[kk], acc.at[didx.at[kk]],
                                  s_s[kk]).wait()

        rows = N // NS
        sl = pl.ds(s * rows, rows)

        @pl.when(c == 0)
        def _():
            pltpu.sync_copy(h0_hbm.at[sl], acc.at[sl])

        @pl.when(c == 1)
        def _():
            pltpu.sync_copy(h1_hbm.at[sl], acc.at[sl])

        for kk in range(4):
            issue_src(kk, kk)
        for kk in range(3):
            issue_dst(kk, kk)
        for kk in range(3):
            wait_src(kk)
            issue_gather(kk)
        plsc.subcore_barrier()

        def chunk_body(i, kk, first, last):
            j = 4 * i + kk
            wait_gather(kk)
            wait_dst(kk)
            issue_scatter(kk)
            if not (first and kk == 0):
                wait_scatter((kk + 3) % 4)
            km = (kk + 3) % 4
            if not last or kk < 1:
                issue_dst(j + 3, km)
            if not last:
                issue_src(j + 4, kk)
            if not last or kk < 1:
                wait_src(km)
                issue_gather(km)

        for kk in range(4):
            chunk_body(0, kk, True, False)

        def step(i, carry):
            for kk in range(4):
                chunk_body(i, kk, False, False)
            return carry

        lax.fori_loop(1, n // 4 - 1, step, 0)
        for kk in range(4):
            chunk_body(n // 4 - 1, kk, False, True)
        wait_scatter(3)
        plsc.subcore_barrier()

        @pl.when(c == 0)
        def _():
            pltpu.sync_copy(acc.at[sl], a0_hbm.at[sl])

        @pl.when(c == 1)
        def _():
            pltpu.sync_copy(acc.at[sl], a1_hbm.at[sl])

    return k(h0, h1, srcpk, dstpk)


def _tc_mm1_fused(x, W1, degpart):
    def body(x_ref, w_ref, p_ref, d_ref, o0_ref, o1_ref):
        p = p_ref[...]
        dinv = lax.rsqrt(p[0] + p[1] + 1.0)
        d_ref[...] = dinv
        h = jnp.dot(x_ref[...], w_ref[...], preferred_element_type=jnp.float32)
        hp = (dinv[:, 0:1] * h).astype(jnp.bfloat16)
        o0_ref[...] = hp[:, :128]
        o1_ref[...] = hp[:, 128:]

    return pl.pallas_call(
        body,
        grid=(N // BM,),
        out_shape=(
            jax.ShapeDtypeStruct((N, LANES), jnp.float32),
            jax.ShapeDtypeStruct((N, 128), jnp.bfloat16),
            jax.ShapeDtypeStruct((N, 128), jnp.bfloat16),
        ),
        in_specs=[
            pl.BlockSpec((BM, 128), lambda i: (i, 0)),
            pl.BlockSpec((128, 256), lambda i: (0, 0)),
            pl.BlockSpec((NC, BM, LANES), lambda i: (0, i, 0)),
        ],
        out_specs=(
            pl.BlockSpec((BM, LANES), lambda i: (i, 0)),
            pl.BlockSpec((BM, 128), lambda i: (i, 0)),
            pl.BlockSpec((BM, 128), lambda i: (i, 0)),
        ),
    )(x, W1, degpart)


def _tc_mm_next(a0, a1, bprev, W, dinv16, fout):
    fh = fout // 2

    def body(a0_ref, a1_ref, b_ref, w_ref, d_ref, o0_ref, o1_ref):
        d1 = d_ref[:, 0:1]
        xin = jnp.concatenate(
            [a0_ref[...], a1_ref[...]], axis=1).astype(jnp.float32)
        pre = d1 * xin + b_ref[...]
        xact = jnp.where(pre > 0, pre, 0.1 * pre)
        h = jnp.dot(xact, w_ref[...], preferred_element_type=jnp.float32)
        hp = (d1 * h).astype(jnp.bfloat16)
        o0_ref[...] = hp[:, :fh]
        o1_ref[...] = hp[:, fh:]

    return pl.pallas_call(
        body,
        grid=(N // BM,),
        out_shape=(
            jax.ShapeDtypeStruct((N, fh), jnp.bfloat16),
            jax.ShapeDtypeStruct((N, fh), jnp.bfloat16),
        ),
        in_specs=[
            pl.BlockSpec((BM, 128), lambda i: (i, 0)),
            pl.BlockSpec((BM, 128), lambda i: (i, 0)),
            pl.BlockSpec((1, 256), lambda i: (0, 0)),
            pl.BlockSpec((256, fout), lambda i: (0, 0)),
            pl.BlockSpec((BM, LANES), lambda i: (i, 0)),
        ],
        out_specs=(
            pl.BlockSpec((BM, fh), lambda i: (i, 0)),
            pl.BlockSpec((BM, fh), lambda i: (i, 0)),
        ),
    )(a0, a1, bprev, W, dinv16)


def _tc_final(a0, a1, b3, dinv16):
    def body(a0_ref, a1_ref, b_ref, d_ref, o_ref):
        d1 = d_ref[:, 0:1]
        xin = jnp.concatenate(
            [a0_ref[...], a1_ref[...]], axis=1).astype(jnp.float32)
        o_ref[...] = d1 * xin + b_ref[...]

    return pl.pallas_call(
        body,
        grid=(N // BM,),
        out_shape=jax.ShapeDtypeStruct((N, 128), jnp.float32),
        in_specs=[
            pl.BlockSpec((BM, 64), lambda i: (i, 0)),
            pl.BlockSpec((BM, 64), lambda i: (i, 0)),
            pl.BlockSpec((1, 128), lambda i: (0, 0)),
            pl.BlockSpec((BM, LANES), lambda i: (i, 0)),
        ],
        out_specs=pl.BlockSpec((BM, 128), lambda i: (i, 0)),
    )(a0, a1, b3, dinv16)


def kernel(x, edge_index, W1, b1, W2, b2, W3, b3):
    src = edge_index[0].astype(jnp.int32)
    dst = edge_index[1].astype(jnp.int32)
    dst_deg = dst.reshape(NC * NS, E // (NC * NS * CHUNK), CHUNK)
    agg_chunk = 100
    nagg = E // (NS * agg_chunk)
    srcpk = src.reshape(NS, nagg, agg_chunk)
    dstpk = dst.reshape(NS, nagg, agg_chunk)
    zeros = jnp.zeros((N, LANES), jnp.float32)
    ones = jnp.ones((CHUNK, LANES), jnp.float32)

    degpart = _sc_degree(dst_deg, zeros, ones)
    dinv16, h0, h1 = _tc_mm1_fused(x, W1, degpart)
    a0, a1 = _sc_aggregate(h0, h1, srcpk, dstpk, 128, agg_chunk)
    h0, h1 = _tc_mm_next(a0, a1, b1.reshape(1, 256), W2, dinv16, 256)
    a0, a1 = _sc_aggregate(h0, h1, srcpk, dstpk, 128, agg_chunk)
    h0, h1 = _tc_mm_next(a0, a1, b2.reshape(1, 256), W3, dinv16, 128)
    a0, a1 = _sc_aggregate(h0, h1, srcpk, dstpk, 64, agg_chunk)
    return _tc_final(a0, a1, b3.reshape(1, 128), dinv16)
